# baseline (speedup 1.0000x reference)
import functools

import jax
import jax.numpy as jnp
from jax import lax
from jax.experimental import pallas as pl
from jax.experimental.pallas import tpu as pltpu
from jax.experimental.pallas import tpu_sc as plsc

_LANES = 16


@functools.lru_cache(maxsize=None)
def _make_kernel(B, L, V, D):
    info = plsc.get_sparse_core_info()
    NC, NS = info.num_cores, info.num_subcores
    NW = NC * NS
    assert L % NW == 0 and D % _LANES == 0
    LW = L // NW
    C = min(32, LW)
    assert LW % C == 0 and C % 8 == 0
    n_chunks = LW // C
    n_vec = D // _LANES
    steps = [(ci, b) for ci in range(n_chunks) for b in range(B)]
    T = len(steps)

    mesh = plsc.VectorSubcoreMesh(core_axis_name="c", subcore_axis_name="s")

    @functools.partial(
        pl.kernel,
        mesh=mesh,
        out_type=jax.ShapeDtypeStruct((B, L, D), jnp.float32),
        scratch_types=[
            pltpu.VMEM((B, LW), jnp.int32),
            pltpu.VMEM((2, C, D), jnp.float32),
            pltpu.VMEM((3, C, D), jnp.float32),
            pltpu.SemaphoreType.DMA,
            pltpu.SemaphoreType.DMA,
            pltpu.SemaphoreType.DMA,
            pltpu.SemaphoreType.DMA,
        ],
    )
    def emb(x_hbm, table_hbm, enc_hbm, out_hbm,
            idx_v, enc_v, rows_v, gsem, esem, isem, ssem):
        wid = lax.axis_index("s") * NC + lax.axis_index("c")
        l0 = wid * LW

        for b in range(B):
            pltpu.async_copy(x_hbm.at[b, pl.ds(l0, LW)], idx_v.at[b], isem)
        for b in range(B):
            pltpu.make_async_copy(
                x_hbm.at[b, pl.ds(l0, LW)], idx_v.at[b], isem
            ).wait()

        def fire(ci, b, slot):
            pltpu.async_copy(
                table_hbm.at[idx_v.at[b, pl.ds(ci * C, C)]],
                rows_v.at[slot],
                gsem,
            )

        def fire_enc(ci):
            pltpu.async_copy(
                enc_hbm.at[pl.ds(l0 + ci * C, C)], enc_v.at[ci % 2], esem
            )

        def wait_enc(ci):
            pltpu.make_async_copy(
                enc_hbm.at[pl.ds(l0 + ci * C, C)], enc_v.at[ci % 2], esem
            ).wait()

        def store(ci, b, slot):
            pltpu.async_copy(
                rows_v.at[slot], out_hbm.at[b, pl.ds(l0 + ci * C, C)], ssem
            )

        def wait_one_store():
            pltpu.make_async_copy(
                rows_v.at[0], out_hbm.at[0, pl.ds(l0, C)], ssem
            ).wait()

        fire_enc(0)
        fire(0, 0, 0)

        stores_fired = 0
        stores_waited = 0
        for t, (ci, b) in enumerate(steps):
            slot = t % 3
            if t + 1 < T:
                if t >= 2:
                    wait_one_store()
                    stores_waited += 1
                fire(steps[t + 1][0], steps[t + 1][1], (t + 1) % 3)
            if b == 0:
                wait_enc(ci)
                if ci + 1 < n_chunks:
                    fire_enc(ci + 1)
            pltpu.make_async_copy(
                table_hbm.at[idx_v.at[b, pl.ds(ci * C, C)]],
                rows_v.at[slot],
                gsem,
            ).wait()

            def row_body(r, _, slot=slot, eslot=ci % 2):
                for j in range(n_vec):
                    sl = pl.ds(j * _LANES, _LANES)
                    rows_v[slot, r, sl] = rows_v[slot, r, sl] + enc_v[eslot, r, sl]
                return 0

            lax.fori_loop(0, C, row_body, 0)
            store(ci, b, slot)
            stores_fired += 1

        for _ in range(stores_fired - stores_waited):
            wait_one_store()

    return emb


def kernel(x, table, enc):
    B, L = x.shape
    V, D = table.shape
    emb = _make_kernel(B, L, V, D)
    return emb(x.astype(jnp.int32), table, enc[:L])


# speedup vs baseline: 1.2835x; 1.2835x over previous
import functools

import jax
import jax.numpy as jnp
from jax import lax
from jax.experimental import pallas as pl
from jax.experimental.pallas import tpu as pltpu
from jax.experimental.pallas import tpu_sc as plsc

_LANES = 16


@functools.lru_cache(maxsize=None)
def _make_kernel(B, L, V, D):
    info = plsc.get_sparse_core_info()
    NC, NS = info.num_cores, info.num_subcores
    NW = NC * NS
    assert L % NW == 0 and D % _LANES == 0
    LW = L // NW
    C = min(32, LW)
    assert LW % C == 0 and C % 8 == 0
    n_chunks = LW // C
    n_vec = D // _LANES
    steps = [(ci, b) for ci in range(n_chunks) for b in range(B)]

    mesh = plsc.VectorSubcoreMesh(core_axis_name="c", subcore_axis_name="s")

    @functools.partial(
        pl.kernel,
        mesh=mesh,
        out_type=jax.ShapeDtypeStruct((B, L, D), jnp.float32),
        scratch_types=[
            pltpu.VMEM((2, C), jnp.int32),
            pltpu.VMEM((C, D), jnp.float32),
            pltpu.VMEM((2, C, D), jnp.float32),
            pltpu.SemaphoreType.DMA,
            pltpu.SemaphoreType.DMA,
        ],
    )
    def emb(x_hbm, table_hbm, enc_hbm, out_hbm, idx_v, enc_v, rows_v, gsem, esem):
        wid = lax.axis_index("s") * NC + lax.axis_index("c")
        l0 = wid * LW

        def fire(ci, b, slot):
            base = l0 + ci * C
            pltpu.sync_copy(x_hbm.at[b, pl.ds(base, C)], idx_v.at[slot])
            pltpu.async_copy(table_hbm.at[idx_v.at[slot]], rows_v.at[slot], gsem)

        pltpu.async_copy(enc_hbm.at[pl.ds(l0, C)], enc_v, esem)
        fire(0, 0, 0)

        for t, (ci, b) in enumerate(steps):
            slot = t % 2
            if t + 1 < len(steps):
                fire(steps[t + 1][0], steps[t + 1][1], (t + 1) % 2)
            if b == 0 and ci > 0:
                pltpu.async_copy(enc_hbm.at[pl.ds(l0 + ci * C, C)], enc_v, esem)
            if b == 0:
                pltpu.make_async_copy(
                    enc_hbm.at[pl.ds(l0, C)], enc_v, esem
                ).wait()
            pltpu.make_async_copy(
                table_hbm.at[idx_v.at[slot]], rows_v.at[slot], gsem
            ).wait()

            def row_body(r, _, slot=slot):
                for j in range(n_vec):
                    sl = pl.ds(j * _LANES, _LANES)
                    plsc.addupdate(rows_v.at[slot, r, sl], enc_v[r, sl])
                return 0

            lax.fori_loop(0, C, row_body, 0)
            pltpu.sync_copy(rows_v.at[slot], out_hbm.at[b, pl.ds(l0 + ci * C, C)])

    return emb


def kernel(x, table, enc):
    B, L = x.shape
    V, D = table.shape
    emb = _make_kernel(B, L, V, D)
    return emb(x.astype(jnp.int32), table, enc[:L])


# speedup vs baseline: 1.4113x; 1.0995x over previous
import functools

import jax
import jax.numpy as jnp
from jax import lax
from jax.experimental import pallas as pl
from jax.experimental.pallas import tpu as pltpu
from jax.experimental.pallas import tpu_sc as plsc

_LANES = 16


@functools.lru_cache(maxsize=None)
def _make_kernel(B, L, V, D):
    info = plsc.get_sparse_core_info()
    NC, NS = info.num_cores, info.num_subcores
    NW = NC * NS
    assert L % NW == 0 and D % _LANES == 0
    LW = L // NW
    C = min(32, LW)
    assert LW % C == 0 and C % 8 == 0
    n_chunks = LW // C
    n_vec = D // _LANES
    steps = [(ci, b) for ci in range(n_chunks) for b in range(B)]

    mesh = plsc.VectorSubcoreMesh(core_axis_name="c", subcore_axis_name="s")

    @functools.partial(
        pl.kernel,
        mesh=mesh,
        out_type=jax.ShapeDtypeStruct((B, L, D), jnp.float32),
        scratch_types=[
            pltpu.VMEM((B, LW), jnp.int32),
            pltpu.VMEM((C, D), jnp.float32),
            pltpu.VMEM((2, C, D), jnp.float32),
            pltpu.SemaphoreType.DMA,
            pltpu.SemaphoreType.DMA,
            pltpu.SemaphoreType.DMA,
        ],
    )
    def emb(x_hbm, table_hbm, enc_hbm, out_hbm,
            idx_v, enc_v, rows_v, gsem, esem, isem):
        wid = lax.axis_index("s") * NC + lax.axis_index("c")
        l0 = wid * LW

        for b in range(B):
            pltpu.async_copy(x_hbm.at[b, pl.ds(l0, LW)], idx_v.at[b], isem)
        for b in range(B):
            pltpu.make_async_copy(
                x_hbm.at[b, pl.ds(l0, LW)], idx_v.at[b], isem
            ).wait()

        def fire(ci, b, slot):
            pltpu.async_copy(
                table_hbm.at[idx_v.at[b, pl.ds(ci * C, C)]],
                rows_v.at[slot],
                gsem,
            )

        pltpu.async_copy(enc_hbm.at[pl.ds(l0, C)], enc_v, esem)
        fire(0, 0, 0)

        for t, (ci, b) in enumerate(steps):
            slot = t % 2
            if t + 1 < len(steps):
                fire(steps[t + 1][0], steps[t + 1][1], (t + 1) % 2)
            if b == 0 and ci > 0:
                pltpu.async_copy(enc_hbm.at[pl.ds(l0 + ci * C, C)], enc_v, esem)
            if b == 0:
                pltpu.make_async_copy(
                    enc_hbm.at[pl.ds(l0, C)], enc_v, esem
                ).wait()
            pltpu.make_async_copy(
                table_hbm.at[idx_v.at[b, pl.ds(ci * C, C)]],
                rows_v.at[slot],
                gsem,
            ).wait()

            def row_body(r, _, slot=slot):
                for j in range(n_vec):
                    sl = pl.ds(j * _LANES, _LANES)
                    plsc.addupdate(rows_v.at[slot, r, sl], enc_v[r, sl])
                return 0

            lax.fori_loop(0, C, row_body, 0)
            pltpu.sync_copy(rows_v.at[slot], out_hbm.at[b, pl.ds(l0 + ci * C, C)])

    return emb


def kernel(x, table, enc):
    B, L = x.shape
    V, D = table.shape
    emb = _make_kernel(B, L, V, D)
    return emb(x.astype(jnp.int32), table, enc)
